# baseline (speedup 1.0000x reference)
import functools

import jax
import jax.numpy as jnp
from jax import lax
from jax.experimental import pallas as pl
from jax.experimental.pallas import tpu as pltpu
from jax.experimental.pallas import tpu_sc as plsc

N_NODES = 100000
EMB_DIM = 128
PRIOR_DIM = 64
OUT_DIM = PRIOR_DIM + EMB_DIM
CHUNK = 128
NUM_FULL = N_NODES // CHUNK
REM = N_NODES - NUM_FULL * CHUNK
NUM_CHUNKS_PAD = NUM_FULL + 1
NW = 32


def _build_kernel():
    mesh = plsc.VectorSubcoreMesh(core_axis_name="c", subcore_axis_name="s")

    @functools.partial(
        pl.kernel,
        mesh=mesh,
        compiler_params=pltpu.CompilerParams(use_tc_tiling_on_sc=False),
        out_type=jax.ShapeDtypeStruct((N_NODES, OUT_DIM), jnp.float32),
        scratch_types=[
            pltpu.VMEM((CHUNK,), jnp.int32),
            pltpu.VMEM((CHUNK, EMB_DIM), jnp.float32),
            pltpu.VMEM((CHUNK, PRIOR_DIM), jnp.float32),
            pltpu.SemaphoreType.DMA,
        ],
    )
    def k(idx_hbm, inv_hbm, tab_hbm, out_hbm, idx_v, emb_v, inv_v, sem):
        cid = lax.axis_index("c")
        sid = lax.axis_index("s")
        wid = sid * 2 + cid
        nloops = jnp.where(wid <= (NUM_FULL - 1) % NW, (NUM_FULL + NW - 1) // NW,
                           NUM_FULL // NW)

        def body(t, carry):
            c = wid + NW * t
            base = c * CHUNK
            pltpu.sync_copy(idx_hbm.at[c], idx_v)
            pltpu.async_copy(tab_hbm.at[idx_v], emb_v, sem).wait()
            pltpu.sync_copy(inv_hbm.at[pl.ds(base, CHUNK), :], inv_v)
            pltpu.sync_copy(inv_v, out_hbm.at[pl.ds(base, CHUNK), pl.ds(0, PRIOR_DIM)])
            pltpu.sync_copy(emb_v, out_hbm.at[pl.ds(base, CHUNK), pl.ds(PRIOR_DIM, EMB_DIM)])
            return carry

        lax.fori_loop(0, nloops, body, 0)

        @pl.when(wid == NW - 1)
        def _tail():
            base = NUM_FULL * CHUNK
            pltpu.sync_copy(idx_hbm.at[NUM_FULL], idx_v)
            pltpu.async_copy(tab_hbm.at[idx_v], emb_v, sem).wait()
            pltpu.sync_copy(inv_hbm.at[pl.ds(base, REM), :], inv_v.at[pl.ds(0, REM)])
            pltpu.sync_copy(inv_v.at[pl.ds(0, REM)],
                            out_hbm.at[pl.ds(base, REM), pl.ds(0, PRIOR_DIM)])
            pltpu.sync_copy(emb_v.at[pl.ds(0, REM)],
                            out_hbm.at[pl.ds(base, REM), pl.ds(PRIOR_DIM, EMB_DIM)])

    return k


_KERNEL = _build_kernel()


def kernel(feature, invariant_node_features, table):
    feat = feature.astype(jnp.int32)
    pad = NUM_CHUNKS_PAD * CHUNK - N_NODES
    feat2d = jnp.pad(feat, (0, pad)).reshape(NUM_CHUNKS_PAD, CHUNK)
    return _KERNEL(feat2d, invariant_node_features, table)
